# Initial kernel scaffold; baseline (speedup 1.0000x reference)
import jax
import jax.numpy as jnp
from jax.experimental import pallas as pl


def kernel(x, patch_w, patch_b, cls_token, pos_embed, norm1_w, norm1_b, attn_in_w, attn_in_b, attn_out_w, attn_out_b, norm2_w, norm2_b, router_w, router_b, e_w1, e_b1, e_w2, e_b2, fnorm_w, fnorm_b, head1_w, head1_b, head2_w, head2_b):
    raise NotImplementedError("write your pallas kernel here")


# speedup vs baseline: 1.7101x; 1.7101x over previous
import functools
import math

import jax
import jax.numpy as jnp
from jax import lax
from jax.experimental import pallas as pl
from jax.experimental.pallas import tpu as pltpu

_L = 2
_D = 256
_E = 8
_HD = 1024
_H = 8
_P = 16
_IMG = 224
_B = 8
_S = (_IMG // _P) ** 2 + 1
_NC = 1000
_T = _B * _S
_DH = _D // _H


def _erf(x):
    a1, a2, a3, a4, a5 = (0.254829592, -0.284496736, 1.421413741,
                          -1.453152027, 1.061405429)
    p = 0.3275911
    s = jnp.sign(x)
    z = jnp.abs(x)
    t = 1.0 / (1.0 + p * z)
    poly = t * (a1 + t * (a2 + t * (a3 + t * (a4 + t * a5))))
    return s * (1.0 - poly * jnp.exp(-z * z))


def _gelu(x):
    return 0.5 * x * (1.0 + _erf(x * (1.0 / math.sqrt(2.0))))


def _ln(x, w, b):
    m = jnp.mean(x, axis=-1, keepdims=True)
    c = x - m
    v = jnp.mean(c * c, axis=-1, keepdims=True)
    return c * jax.lax.rsqrt(v + 1e-5) * w + b


def _patch_kernel(patches_ref, w_ref, b_ref, cls_ref, pos_ref, out_ref):
    pe = jnp.dot(patches_ref[0], w_ref[...],
                 preferred_element_type=jnp.float32) + b_ref[...]
    out_ref[0, :1, :] = cls_ref[...] + pos_ref[:1, :]
    out_ref[0, 1:, :] = pe + pos_ref[1:, :]


def _patch_embed(patches, w_t, b, cls, pos):
    return pl.pallas_call(
        _patch_kernel,
        grid=(_B,),
        in_specs=[
            pl.BlockSpec((1, _S - 1, 3 * _P * _P), lambda i: (i, 0, 0)),
            pl.BlockSpec((3 * _P * _P, _D), lambda i: (0, 0)),
            pl.BlockSpec((1, _D), lambda i: (0, 0)),
            pl.BlockSpec((1, _D), lambda i: (0, 0)),
            pl.BlockSpec((_S, _D), lambda i: (0, 0)),
        ],
        out_specs=pl.BlockSpec((1, _S, _D), lambda i: (i, 0, 0)),
        out_shape=jax.ShapeDtypeStruct((_B, _S, _D), jnp.float32),
    )(patches, w_t, b, cls, pos)


def _attn_kernel(h_ref, n1w_ref, n1b_ref, wqkv_ref, bqkv_ref, wout_ref,
                 bout_ref, out_ref):
    x = h_ref[0]
    x2 = _ln(x, n1w_ref[...], n1b_ref[...])
    qkv = jnp.dot(x2, wqkv_ref[...],
                  preferred_element_type=jnp.float32) + bqkv_ref[...]
    scale = 1.0 / math.sqrt(float(_DH))
    outs = []
    for hh in range(_H):
        q = qkv[:, hh * _DH:(hh + 1) * _DH]
        k = qkv[:, _D + hh * _DH:_D + (hh + 1) * _DH]
        v = qkv[:, 2 * _D + hh * _DH:2 * _D + (hh + 1) * _DH]
        s = lax.dot_general(q, k, (((1,), (1,)), ((), ())),
                            preferred_element_type=jnp.float32) * scale
        m = jnp.max(s, axis=-1, keepdims=True)
        p = jnp.exp(s - m)
        p = p / jnp.sum(p, axis=-1, keepdims=True)
        outs.append(jnp.dot(p, v, preferred_element_type=jnp.float32))
    o = jnp.concatenate(outs, axis=1)
    proj = jnp.dot(o, wout_ref[...],
                   preferred_element_type=jnp.float32) + bout_ref[...]
    out_ref[0] = x + proj


def _attn_block(h, n1w, n1b, wqkv_t, bqkv, wout_t, bout):
    return pl.pallas_call(
        _attn_kernel,
        grid=(_B,),
        in_specs=[
            pl.BlockSpec((1, _S, _D), lambda i: (i, 0, 0)),
            pl.BlockSpec((1, _D), lambda i: (0, 0)),
            pl.BlockSpec((1, _D), lambda i: (0, 0)),
            pl.BlockSpec((_D, 3 * _D), lambda i: (0, 0)),
            pl.BlockSpec((1, 3 * _D), lambda i: (0, 0)),
            pl.BlockSpec((_D, _D), lambda i: (0, 0)),
            pl.BlockSpec((1, _D), lambda i: (0, 0)),
        ],
        out_specs=pl.BlockSpec((1, _S, _D), lambda i: (i, 0, 0)),
        out_shape=jax.ShapeDtypeStruct((_B, _S, _D), jnp.float32),
    )(h, n1w, n1b, wqkv_t, bqkv, wout_t, bout)


def _moe_kernel(h_ref, n2w_ref, n2b_ref, rw_ref, rb_ref, w1_ref, b1_ref,
                w2_ref, b2_ref, out_ref, x3_s, gates_s):
    e = pl.program_id(0)

    @pl.when(e == 0)
    def _init():
        x3 = _ln(h_ref[...], n2w_ref[...], n2b_ref[...])
        x3_s[...] = x3
        logits = jnp.dot(x3, rw_ref[...],
                         preferred_element_type=jnp.float32) + rb_ref[...]
        lm = jnp.max(logits, axis=-1, keepdims=True)
        pe_ = jnp.exp(logits - lm)
        probs = pe_ / jnp.sum(pe_, axis=-1, keepdims=True)
        idx = lax.broadcasted_iota(jnp.int32, (_T, _E), 1)
        m1 = jnp.max(probs, axis=-1, keepdims=True)
        e1 = jnp.min(jnp.where(probs == m1, idx, _E), axis=-1, keepdims=True)
        oh1 = (idx == e1).astype(jnp.float32)
        probs2 = jnp.where(idx == e1, -jnp.inf, probs)
        m2 = jnp.max(probs2, axis=-1, keepdims=True)
        e2 = jnp.min(jnp.where(probs2 == m2, idx, _E), axis=-1, keepdims=True)
        oh2 = (idx == e2).astype(jnp.float32)
        gates_s[...] = (m1 * oh1 + m2 * oh2) / (m1 + m2)
        out_ref[...] = h_ref[...]

    x3 = x3_s[...]
    h1 = _gelu(jnp.dot(x3, w1_ref[0],
                       preferred_element_type=jnp.float32) + b1_ref[0])
    y = jnp.dot(h1, w2_ref[0],
                preferred_element_type=jnp.float32) + b2_ref[0]
    idx = lax.broadcasted_iota(jnp.int32, (_T, _E), 1)
    g = jnp.sum(jnp.where(idx == e, gates_s[...], 0.0), axis=-1,
                keepdims=True)
    out_ref[...] += g * y


def _moe_block(hflat, n2w, n2b, rw_t, rb, w1, b1, w2, b2):
    return pl.pallas_call(
        _moe_kernel,
        grid=(_E,),
        in_specs=[
            pl.BlockSpec((_T, _D), lambda e: (0, 0)),
            pl.BlockSpec((1, _D), lambda e: (0, 0)),
            pl.BlockSpec((1, _D), lambda e: (0, 0)),
            pl.BlockSpec((_D, _E), lambda e: (0, 0)),
            pl.BlockSpec((1, _E), lambda e: (0, 0)),
            pl.BlockSpec((1, _D, _HD), lambda e: (e, 0, 0)),
            pl.BlockSpec((1, 1, _HD), lambda e: (e, 0, 0)),
            pl.BlockSpec((1, _HD, _D), lambda e: (e, 0, 0)),
            pl.BlockSpec((1, 1, _D), lambda e: (e, 0, 0)),
        ],
        out_specs=pl.BlockSpec((_T, _D), lambda e: (0, 0)),
        out_shape=jax.ShapeDtypeStruct((_T, _D), jnp.float32),
        scratch_shapes=[
            pltpu.VMEM((_T, _D), jnp.float32),
            pltpu.VMEM((_T, _E), jnp.float32),
        ],
        compiler_params=pltpu.CompilerParams(
            dimension_semantics=("arbitrary",)),
    )(hflat, n2w, n2b, rw_t, rb, w1, b1, w2, b2)


def _head_kernel(cls_ref, fw_ref, fb_ref, w1_ref, b1_ref, w2_ref, b2_ref,
                 out_ref):
    c = _ln(cls_ref[...], fw_ref[...], fb_ref[...])
    z = _gelu(jnp.dot(c, w1_ref[...],
                      preferred_element_type=jnp.float32) + b1_ref[...])
    out_ref[...] = jnp.dot(z, w2_ref[...],
                           preferred_element_type=jnp.float32) + b2_ref[...]


def _head(cls_rows, fw, fb, h1w_t, h1b, h2w_t, h2b):
    return pl.pallas_call(
        _head_kernel,
        in_specs=[
            pl.BlockSpec((_B, _D), lambda: (0, 0)),
            pl.BlockSpec((1, _D), lambda: (0, 0)),
            pl.BlockSpec((1, _D), lambda: (0, 0)),
            pl.BlockSpec((_D, _D), lambda: (0, 0)),
            pl.BlockSpec((1, _D), lambda: (0, 0)),
            pl.BlockSpec((_D, _NC), lambda: (0, 0)),
            pl.BlockSpec((1, _NC), lambda: (0, 0)),
        ],
        out_specs=pl.BlockSpec((_B, _NC), lambda: (0, 0)),
        out_shape=jax.ShapeDtypeStruct((_B, _NC), jnp.float32),
    )(cls_rows, fw, fb, h1w_t, h1b, h2w_t, h2b)


def kernel(x, patch_w, patch_b, cls_token, pos_embed, norm1_w, norm1_b,
           attn_in_w, attn_in_b, attn_out_w, attn_out_b, norm2_w, norm2_b,
           router_w, router_b, e_w1, e_b1, e_w2, e_b2, fnorm_w, fnorm_b,
           head1_w, head1_b, head2_w, head2_b):
    nP = _IMG // _P
    patches = x.reshape(_B, 3, nP, _P, nP, _P)
    patches = patches.transpose(0, 2, 4, 1, 3, 5).reshape(_B, nP * nP,
                                                          3 * _P * _P)
    pw_t = patch_w.reshape(_D, 3 * _P * _P).T

    h = _patch_embed(patches, pw_t, patch_b.reshape(1, _D),
                     cls_token.reshape(1, _D), pos_embed.reshape(_S, _D))

    for i in range(_L):
        h = _attn_block(
            h,
            norm1_w[i].reshape(1, _D), norm1_b[i].reshape(1, _D),
            attn_in_w[i].T, attn_in_b[i].reshape(1, 3 * _D),
            attn_out_w[i].T, attn_out_b[i].reshape(1, _D),
        )
        hflat = h.reshape(_T, _D)
        hflat = _moe_block(
            hflat,
            norm2_w[i].reshape(1, _D), norm2_b[i].reshape(1, _D),
            router_w[i].T, router_b[i].reshape(1, _E),
            e_w1[i], e_b1[i].reshape(_E, 1, _HD),
            e_w2[i], e_b2[i].reshape(_E, 1, _D),
        )
        h = hflat.reshape(_B, _S, _D)

    cls_rows = h[:, 0, :]
    return _head(cls_rows, fnorm_w.reshape(1, _D), fnorm_b.reshape(1, _D),
                 head1_w.T, head1_b.reshape(1, _D),
                 head2_w.T, head2_b.reshape(1, _NC))


# speedup vs baseline: 1.7427x; 1.0190x over previous
import functools
import math

import jax
import jax.numpy as jnp
from jax import lax
from jax.experimental import pallas as pl
from jax.experimental.pallas import tpu as pltpu

_L = 2
_D = 256
_E = 8
_HD = 1024
_H = 8
_P = 16
_IMG = 224
_B = 8
_S = (_IMG // _P) ** 2 + 1
_NC = 1000
_T = _B * _S
_DH = _D // _H


def _erf(x):
    a1, a2, a3, a4, a5 = (0.254829592, -0.284496736, 1.421413741,
                          -1.453152027, 1.061405429)
    p = 0.3275911
    s = jnp.sign(x)
    z = jnp.abs(x)
    t = 1.0 / (1.0 + p * z)
    poly = t * (a1 + t * (a2 + t * (a3 + t * (a4 + t * a5))))
    return s * (1.0 - poly * jnp.exp(-z * z))


def _gelu(x):
    return 0.5 * x * (1.0 + _erf(x * (1.0 / math.sqrt(2.0))))


def _ln(x, w, b):
    m = jnp.mean(x, axis=-1, keepdims=True)
    c = x - m
    v = jnp.mean(c * c, axis=-1, keepdims=True)
    return c * jax.lax.rsqrt(v + 1e-5) * w + b


def _patch_kernel(patches_ref, w_ref, b_ref, cls_ref, pos_ref, out_ref):
    pe = jnp.dot(patches_ref[0], w_ref[...],
                 preferred_element_type=jnp.float32) + b_ref[...]
    out_ref[0, :1, :] = cls_ref[...] + pos_ref[:1, :]
    out_ref[0, 1:, :] = pe + pos_ref[1:, :]


def _patch_embed(patches, w_t, b, cls, pos):
    return pl.pallas_call(
        _patch_kernel,
        grid=(_B,),
        in_specs=[
            pl.BlockSpec((1, _S - 1, 3 * _P * _P), lambda i: (i, 0, 0)),
            pl.BlockSpec((3 * _P * _P, _D), lambda i: (0, 0)),
            pl.BlockSpec((1, _D), lambda i: (0, 0)),
            pl.BlockSpec((1, _D), lambda i: (0, 0)),
            pl.BlockSpec((_S, _D), lambda i: (0, 0)),
        ],
        out_specs=pl.BlockSpec((1, _S, _D), lambda i: (i, 0, 0)),
        out_shape=jax.ShapeDtypeStruct((_B, _S, _D), jnp.float32),
    )(patches, w_t, b, cls, pos)


def _attn_kernel(h_ref, n1w_ref, n1b_ref, wqkv_ref, bqkv_ref, wout_ref,
                 bout_ref, out_ref):
    x = h_ref[0]
    x2 = _ln(x, n1w_ref[...], n1b_ref[...])
    qkv = jnp.dot(x2, wqkv_ref[...],
                  preferred_element_type=jnp.float32) + bqkv_ref[...]
    scale = 1.0 / math.sqrt(float(_DH))
    outs = []
    for hh in range(_H):
        q = qkv[:, hh * _DH:(hh + 1) * _DH]
        k = qkv[:, _D + hh * _DH:_D + (hh + 1) * _DH]
        v = qkv[:, 2 * _D + hh * _DH:2 * _D + (hh + 1) * _DH]
        s = lax.dot_general(q, k, (((1,), (1,)), ((), ())),
                            preferred_element_type=jnp.float32) * scale
        m = jnp.max(s, axis=-1, keepdims=True)
        p = jnp.exp(s - m)
        p = p / jnp.sum(p, axis=-1, keepdims=True)
        outs.append(jnp.dot(p, v, preferred_element_type=jnp.float32))
    o = jnp.concatenate(outs, axis=1)
    proj = jnp.dot(o, wout_ref[...],
                   preferred_element_type=jnp.float32) + bout_ref[...]
    out_ref[0] = x + proj


def _attn_block(h, n1w, n1b, wqkv_t, bqkv, wout_t, bout):
    return pl.pallas_call(
        _attn_kernel,
        grid=(_B,),
        in_specs=[
            pl.BlockSpec((1, _S, _D), lambda i: (i, 0, 0)),
            pl.BlockSpec((1, _D), lambda i: (0, 0)),
            pl.BlockSpec((1, _D), lambda i: (0, 0)),
            pl.BlockSpec((_D, 3 * _D), lambda i: (0, 0)),
            pl.BlockSpec((1, 3 * _D), lambda i: (0, 0)),
            pl.BlockSpec((_D, _D), lambda i: (0, 0)),
            pl.BlockSpec((1, _D), lambda i: (0, 0)),
        ],
        out_specs=pl.BlockSpec((1, _S, _D), lambda i: (i, 0, 0)),
        out_shape=jax.ShapeDtypeStruct((_B, _S, _D), jnp.float32),
    )(h, n1w, n1b, wqkv_t, bqkv, wout_t, bout)


def _moe_kernel(h_ref, n2w_ref, n2b_ref, rw_ref, rb_ref, w1_ref, b1_ref,
                w2_ref, b2_ref, out_ref, x3_s, gates_s):
    e = pl.program_id(0)

    @pl.when(e == 0)
    def _init():
        x3 = _ln(h_ref[...], n2w_ref[...], n2b_ref[...])
        x3_s[...] = x3
        logits = jnp.dot(x3, rw_ref[...],
                         preferred_element_type=jnp.float32) + rb_ref[...]
        lm = jnp.max(logits, axis=-1, keepdims=True)
        pe_ = jnp.exp(logits - lm)
        probs = pe_ / jnp.sum(pe_, axis=-1, keepdims=True)
        idx = lax.broadcasted_iota(jnp.int32, (_T, _E), 1)
        m1 = jnp.max(probs, axis=-1, keepdims=True)
        e1 = jnp.min(jnp.where(probs == m1, idx, _E), axis=-1, keepdims=True)
        oh1 = (idx == e1).astype(jnp.float32)
        probs2 = jnp.where(idx == e1, -jnp.inf, probs)
        m2 = jnp.max(probs2, axis=-1, keepdims=True)
        e2 = jnp.min(jnp.where(probs2 == m2, idx, _E), axis=-1, keepdims=True)
        oh2 = (idx == e2).astype(jnp.float32)
        gates_s[...] = (m1 * oh1 + m2 * oh2) / (m1 + m2)
        out_ref[...] = h_ref[...]

    x3 = x3_s[...]
    h1 = _gelu(jnp.dot(x3.astype(jnp.bfloat16), w1_ref[0],
                       preferred_element_type=jnp.float32) + b1_ref[0])
    y = jnp.dot(h1.astype(jnp.bfloat16), w2_ref[0],
                preferred_element_type=jnp.float32) + b2_ref[0]
    idx = lax.broadcasted_iota(jnp.int32, (_T, _E), 1)
    g = jnp.sum(jnp.where(idx == e, gates_s[...], 0.0), axis=-1,
                keepdims=True)
    out_ref[...] += g * y


def _moe_block(hflat, n2w, n2b, rw_t, rb, w1, b1, w2, b2):
    return pl.pallas_call(
        _moe_kernel,
        grid=(_E,),
        in_specs=[
            pl.BlockSpec((_T, _D), lambda e: (0, 0)),
            pl.BlockSpec((1, _D), lambda e: (0, 0)),
            pl.BlockSpec((1, _D), lambda e: (0, 0)),
            pl.BlockSpec((_D, _E), lambda e: (0, 0)),
            pl.BlockSpec((1, _E), lambda e: (0, 0)),
            pl.BlockSpec((1, _D, _HD), lambda e: (e, 0, 0)),
            pl.BlockSpec((1, 1, _HD), lambda e: (e, 0, 0)),
            pl.BlockSpec((1, _HD, _D), lambda e: (e, 0, 0)),
            pl.BlockSpec((1, 1, _D), lambda e: (e, 0, 0)),
        ],
        out_specs=pl.BlockSpec((_T, _D), lambda e: (0, 0)),
        out_shape=jax.ShapeDtypeStruct((_T, _D), jnp.float32),
        scratch_shapes=[
            pltpu.VMEM((_T, _D), jnp.float32),
            pltpu.VMEM((_T, _E), jnp.float32),
        ],
        compiler_params=pltpu.CompilerParams(
            dimension_semantics=("arbitrary",)),
    )(hflat, n2w, n2b, rw_t, rb, w1, b1, w2, b2)


def _head_kernel(cls_ref, fw_ref, fb_ref, w1_ref, b1_ref, w2_ref, b2_ref,
                 out_ref):
    c = _ln(cls_ref[...], fw_ref[...], fb_ref[...])
    z = _gelu(jnp.dot(c, w1_ref[...],
                      preferred_element_type=jnp.float32) + b1_ref[...])
    out_ref[...] = jnp.dot(z, w2_ref[...],
                           preferred_element_type=jnp.float32) + b2_ref[...]


def _head(cls_rows, fw, fb, h1w_t, h1b, h2w_t, h2b):
    return pl.pallas_call(
        _head_kernel,
        in_specs=[
            pl.BlockSpec((_B, _D), lambda: (0, 0)),
            pl.BlockSpec((1, _D), lambda: (0, 0)),
            pl.BlockSpec((1, _D), lambda: (0, 0)),
            pl.BlockSpec((_D, _D), lambda: (0, 0)),
            pl.BlockSpec((1, _D), lambda: (0, 0)),
            pl.BlockSpec((_D, _NC), lambda: (0, 0)),
            pl.BlockSpec((1, _NC), lambda: (0, 0)),
        ],
        out_specs=pl.BlockSpec((_B, _NC), lambda: (0, 0)),
        out_shape=jax.ShapeDtypeStruct((_B, _NC), jnp.float32),
    )(cls_rows, fw, fb, h1w_t, h1b, h2w_t, h2b)


def kernel(x, patch_w, patch_b, cls_token, pos_embed, norm1_w, norm1_b,
           attn_in_w, attn_in_b, attn_out_w, attn_out_b, norm2_w, norm2_b,
           router_w, router_b, e_w1, e_b1, e_w2, e_b2, fnorm_w, fnorm_b,
           head1_w, head1_b, head2_w, head2_b):
    nP = _IMG // _P
    patches = x.reshape(_B, 3, nP, _P, nP, _P)
    patches = patches.transpose(0, 2, 4, 1, 3, 5).reshape(_B, nP * nP,
                                                          3 * _P * _P)
    pw_t = patch_w.reshape(_D, 3 * _P * _P).T

    h = _patch_embed(patches, pw_t, patch_b.reshape(1, _D),
                     cls_token.reshape(1, _D), pos_embed.reshape(_S, _D))

    for i in range(_L):
        h = _attn_block(
            h,
            norm1_w[i].reshape(1, _D), norm1_b[i].reshape(1, _D),
            attn_in_w[i].T, attn_in_b[i].reshape(1, 3 * _D),
            attn_out_w[i].T, attn_out_b[i].reshape(1, _D),
        )
        hflat = h.reshape(_T, _D)
        hflat = _moe_block(
            hflat,
            norm2_w[i].reshape(1, _D), norm2_b[i].reshape(1, _D),
            router_w[i].T, router_b[i].reshape(1, _E),
            e_w1[i].astype(jnp.bfloat16), e_b1[i].reshape(_E, 1, _HD),
            e_w2[i].astype(jnp.bfloat16), e_b2[i].reshape(_E, 1, _D),
        )
        h = hflat.reshape(_B, _S, _D)

    cls_rows = h[:, 0, :]
    return _head(cls_rows, fnorm_w.reshape(1, _D), fnorm_b.reshape(1, _D),
                 head1_w.T, head1_b.reshape(1, _D),
                 head2_w.T, head2_b.reshape(1, _NC))
